# baseline (speedup 1.0000x reference)
import functools

import jax
import jax.numpy as jnp
from jax import lax
from jax.experimental import pallas as pl
from jax.experimental.pallas import tpu as pltpu

_NCHUNK = 4


def _manual_kernel(e0_hbm, e1_hbm, w_hbm, b_ref, c_hbm,
                   mean_hbm, cov_hbm,
                   E0, E1, W, CB, M, CV, sems):
    i = pl.program_id(0)
    H, d0 = E0.shape
    d1 = E1.shape[1]
    d = CB.shape[0]
    hd = CV.shape[0]
    ch = H // _NCHUNK

    cw = pltpu.make_async_copy(w_hbm, W, sems.at[0])
    cw.start()
    cc = pltpu.make_async_copy(c_hbm, CB, sems.at[1])
    cc.start()
    ce0 = [pltpu.make_async_copy(
        e0_hbm.at[pl.ds(i * H + t * ch, ch), :],
        E0.at[pl.ds(t * ch, ch), :], sems.at[2 + t]) for t in range(_NCHUNK)]
    ce1 = [pltpu.make_async_copy(
        e1_hbm.at[pl.ds(i * H + t * ch, ch), :],
        E1.at[pl.ds(t * ch, ch), :],
        sems.at[2 + _NCHUNK + t]) for t in range(_NCHUNK)]
    for t in range(_NCHUNK):
        ce0[t].start()
        ce1[t].start()

    cw.wait()
    for t in range(_NCHUNK):
        ce0[t].wait()
        ce1[t].wait()
        rows = pl.ds(t * ch, ch)
        acc = jnp.dot(E0[rows, :], W[0:d0, :],
                      preferred_element_type=jnp.float32)
        acc = acc + jnp.dot(E1[rows, :], W[d0:d0 + d1, :],
                            preferred_element_type=jnp.float32)
        M[rows, :] = acc + b_ref[...]

    cc.wait()
    rr = lax.broadcasted_iota(jnp.int32, (d, d), 0)
    cols = lax.broadcasted_iota(jnp.int32, (d, d), 1)
    CB[...] = jnp.where(cols <= rr, CB[...], jnp.float32(0.0))
    llt = lax.dot_general(
        CB[pl.ds(i * hd, hd), :], CB[...],
        dimension_numbers=(((1,), (1,)), ((), ())),
        preferred_element_type=jnp.float32)
    grow = i * hd + lax.broadcasted_iota(jnp.int32, (hd, d), 0)
    gcol = lax.broadcasted_iota(jnp.int32, (hd, d), 1)
    jitter = jnp.where(grow == gcol, jnp.float32(1e-8), jnp.float32(0.0))
    CV[...] = jnp.maximum(llt + jitter, 0.0)

    cm = pltpu.make_async_copy(M, mean_hbm.at[pl.ds(i * H, H), :],
                               sems.at[2 + 2 * _NCHUNK])
    cm.start()
    cv = pltpu.make_async_copy(CV, cov_hbm.at[pl.ds(i * hd, hd), :],
                               sems.at[3 + 2 * _NCHUNK])
    cv.start()
    cm.wait()
    cv.wait()


def kernel(evidence_0, evidence_1, wt, b, cov_param):
    B, d0 = evidence_0.shape
    d1 = evidence_1.shape[1]
    data_dim = cov_param.shape[0]
    Dp = wt.shape[1]
    H = B // 2
    hd = data_dim // 2

    e0 = evidence_0.astype(jnp.float32)
    e1 = evidence_1.astype(jnp.float32)
    w = wt.astype(jnp.float32)
    bb = b.astype(jnp.float32)
    C = cov_param.astype(jnp.float32)

    mean, cov = pl.pallas_call(
        _manual_kernel,
        out_shape=(
            jax.ShapeDtypeStruct((B, Dp), jnp.float32),
            jax.ShapeDtypeStruct((data_dim, data_dim), jnp.float32),
        ),
        grid=(2,),
        in_specs=[
            pl.BlockSpec(memory_space=pl.ANY),
            pl.BlockSpec(memory_space=pl.ANY),
            pl.BlockSpec(memory_space=pl.ANY),
            pl.BlockSpec((1, Dp), lambda i: (0, 0)),
            pl.BlockSpec(memory_space=pl.ANY),
        ],
        out_specs=(
            pl.BlockSpec(memory_space=pl.ANY),
            pl.BlockSpec(memory_space=pl.ANY),
        ),
        scratch_shapes=[
            pltpu.VMEM((H, d0), jnp.float32),
            pltpu.VMEM((H, d1), jnp.float32),
            pltpu.VMEM((d0 + d1, Dp), jnp.float32),
            pltpu.VMEM((data_dim, data_dim), jnp.float32),
            pltpu.VMEM((H, Dp), jnp.float32),
            pltpu.VMEM((hd, data_dim), jnp.float32),
            pltpu.SemaphoreType.DMA((4 + 2 * _NCHUNK,)),
        ],
        compiler_params=pltpu.CompilerParams(
            dimension_semantics=("parallel",),
            vmem_limit_bytes=60 * 1024 * 1024),
        cost_estimate=pl.CostEstimate(
            flops=2 * B * (d0 + d1) * Dp + 2 * data_dim ** 3,
            transcendentals=0,
            bytes_accessed=4 * (B * (d0 + d1) + B * Dp + (d0 + d1) * Dp
                                + 2 * data_dim * data_dim)),
    )(e0, e1, w, bb, C)

    return mean[:, :data_dim], cov


# speedup vs baseline: 1.3172x; 1.3172x over previous
import functools

import jax
import jax.numpy as jnp
from jax import lax
from jax.experimental import pallas as pl
from jax.experimental.pallas import tpu as pltpu


def _fused_kernel(rb, grid, e0_ref, e1_ref, w0_ref, w1_ref, b_ref, c_ref,
                  mean_ref, cov_ref, l_ref):
    i = pl.program_id(0)
    d = c_ref.shape[0]

    @pl.when((i == 0) | (i == grid // 2))
    def _mask():
        rows = lax.broadcasted_iota(jnp.int32, (d, d), 0)
        cols = lax.broadcasted_iota(jnp.int32, (d, d), 1)
        l_ref[...] = jnp.where(cols <= rows, c_ref[...], jnp.float32(0.0))

    acc = jnp.dot(e0_ref[...], w0_ref[...],
                  preferred_element_type=jnp.float32)
    acc = acc + jnp.dot(e1_ref[...], w1_ref[...],
                        preferred_element_type=jnp.float32)
    mean_ref[...] = acc + b_ref[...]

    llt = lax.dot_general(
        l_ref[pl.ds(i * rb, rb), :], l_ref[...],
        dimension_numbers=(((1,), (1,)), ((), ())),
        preferred_element_type=jnp.float32)
    rows = i * rb + lax.broadcasted_iota(jnp.int32, (rb, d), 0)
    cols = lax.broadcasted_iota(jnp.int32, (rb, d), 1)
    jitter = jnp.where(rows == cols, jnp.float32(1e-8), jnp.float32(0.0))
    cov_ref[...] = jnp.maximum(llt + jitter, 0.0)


def kernel(evidence_0, evidence_1, wt, b, cov_param):
    B, d0 = evidence_0.shape
    d1 = evidence_1.shape[1]
    data_dim = cov_param.shape[0]
    Dp = wt.shape[1]

    grid = 4
    while grid > 1 and (B % grid or data_dim % grid):
        grid //= 2
    TB = B // grid
    rb = data_dim // grid

    e0 = evidence_0.astype(jnp.float32)
    e1 = evidence_1.astype(jnp.float32)
    w = wt.astype(jnp.float32)
    bb = b.astype(jnp.float32)
    C = cov_param.astype(jnp.float32)

    mean, cov = pl.pallas_call(
        functools.partial(_fused_kernel, rb, grid),
        out_shape=(
            jax.ShapeDtypeStruct((B, Dp), jnp.float32),
            jax.ShapeDtypeStruct((data_dim, data_dim), jnp.float32),
        ),
        grid=(grid,),
        in_specs=[
            pl.BlockSpec((TB, d0), lambda i: (i, 0)),
            pl.BlockSpec((TB, d1), lambda i: (i, 0)),
            pl.BlockSpec((d0, Dp), lambda i: (0, 0)),
            pl.BlockSpec((d1, Dp), lambda i: (1, 0)),
            pl.BlockSpec((1, Dp), lambda i: (0, 0)),
            pl.BlockSpec((data_dim, data_dim), lambda i: (0, 0)),
        ],
        out_specs=(
            pl.BlockSpec((TB, Dp), lambda i: (i, 0)),
            pl.BlockSpec((rb, data_dim), lambda i: (i, 0)),
        ),
        scratch_shapes=[pltpu.VMEM((data_dim, data_dim), jnp.float32)],
        compiler_params=pltpu.CompilerParams(
            dimension_semantics=("parallel",)),
        cost_estimate=pl.CostEstimate(
            flops=2 * B * (d0 + d1) * Dp + 2 * data_dim ** 3,
            transcendentals=0,
            bytes_accessed=4 * (B * (d0 + d1) + B * Dp + (d0 + d1) * Dp
                                + 2 * data_dim * data_dim)),
    )(e0, e1, w, w, bb, C)

    return mean[:, :data_dim], cov
